# baseline (speedup 1.0000x reference)
import functools

import jax
import jax.numpy as jnp
from jax import lax
from jax.experimental import pallas as pl
from jax.experimental.pallas import tpu as pltpu
from jax.experimental.pallas import tpu_sc as plsc

NC = 2
NS = 16
NW = NC * NS
D = 32
DP = 128
HP = 56
CH = 256


@functools.lru_cache(maxsize=None)
def _gather_kernel(N):
    rows_w = N // NW
    n_chunks = rows_w // CH
    mesh = plsc.VectorSubcoreMesh(
        core_axis_name="c", subcore_axis_name="s",
        num_cores=NC, num_subcores=NS)

    @functools.partial(
        pl.kernel,
        out_type=jax.ShapeDtypeStruct((N, DP), jnp.float32),
        mesh=mesh,
        scratch_types=[
            pltpu.VMEM((rows_w,), jnp.int32),
            pltpu.VMEM((CH, DP), jnp.float32),
            pltpu.VMEM((CH, DP), jnp.float32),
            pltpu.SemaphoreType.DMA,
            pltpu.SemaphoreType.DMA,
            pltpu.SemaphoreType.DMA,
        ],
        compiler_params=pltpu.CompilerParams(
            use_tc_tiling_on_sc=True, needs_layout_passes=False),
    )
    def k(idx_hbm, table_hbm, out_hbm, idx_v, gb0, gb1, gs0, gs1, isem):
        wid = lax.axis_index("s") * NC + lax.axis_index("c")
        r0 = wid * rows_w

        pltpu.async_copy(
            idx_hbm.at[pl.ds(r0, rows_w)], idx_v, isem).wait()

        def gather_start(c, gb, gs):
            cc = jnp.minimum(c, n_chunks - 1)
            return pltpu.async_copy(
                table_hbm.at[idx_v.at[pl.ds(cc * CH, CH)]], gb, gs)

        def writeback(c, gb):
            pltpu.sync_copy(gb, out_hbm.at[pl.ds(r0 + c * CH, CH)])

        gather_start(0, gb0, gs0)

        def body(s, carry):
            c = 2 * s
            gather_start(c + 1, gb1, gs1)
            pltpu.make_async_copy(
                table_hbm.at[pl.ds(0, CH)], gb0, gs0).wait()
            writeback(c, gb0)
            gather_start(c + 2, gb0, gs0)
            pltpu.make_async_copy(
                table_hbm.at[pl.ds(0, CH)], gb1, gs1).wait()
            writeback(c + 1, gb1)
            return carry

        lax.fori_loop(0, n_chunks // 2, body, 0)
        pltpu.make_async_copy(table_hbm.at[pl.ds(0, CH)], gb0, gs0).wait()
        if n_chunks % 2:
            writeback(n_chunks - 1, gb0)

    return k


def kernel(inputs, embedding_weight):
    B, H = inputs.shape
    table_p = jnp.pad(embedding_weight, ((0, 0), (0, DP - D)))
    idx_p = jnp.pad(inputs.astype(jnp.int32),
                    ((0, 0), (0, HP - H))).reshape(B * HP)
    out_f = _gather_kernel(B * HP)(idx_p, table_p)
    return out_f.reshape(B, HP, DP)[:, :H, :D]


# speedup vs baseline: 5.4838x; 5.4838x over previous
import functools

import jax
import jax.numpy as jnp
from jax import lax
from jax.experimental import pallas as pl
from jax.experimental.pallas import tpu as pltpu
from jax.experimental.pallas import tpu_sc as plsc

NC = 2
NS = 16
NW = NC * NS
D = 32
DP = 128
HP = 56
CH = 256


@functools.lru_cache(maxsize=None)
def _gather_kernel(N):
    rows_w = N // NW
    n_chunks = rows_w // CH
    mesh = plsc.VectorSubcoreMesh(
        core_axis_name="c", subcore_axis_name="s",
        num_cores=NC, num_subcores=NS)

    @functools.partial(
        pl.kernel,
        out_type=jax.ShapeDtypeStruct((N, DP), jnp.float32),
        mesh=mesh,
        scratch_types=[
            pltpu.VMEM((rows_w,), jnp.int32),
            pltpu.VMEM((CH, DP), jnp.float32),
            pltpu.VMEM((CH, DP), jnp.float32),
            pltpu.SemaphoreType.DMA,
            pltpu.SemaphoreType.DMA,
            pltpu.SemaphoreType.DMA,
        ],
        compiler_params=pltpu.CompilerParams(
            use_tc_tiling_on_sc=True, needs_layout_passes=False),
    )
    def k(idx_hbm, table_hbm, out_hbm, idx_v, gb0, gb1, gs0, gs1, isem):
        wid = lax.axis_index("s") * NC + lax.axis_index("c")
        r0 = wid * rows_w

        pltpu.async_copy(
            idx_hbm.at[pl.ds(r0, rows_w)], idx_v, isem).wait()

        def gather_start(c, gb, gs):
            cc = jnp.minimum(c, n_chunks - 1)
            return pltpu.async_copy(
                table_hbm.at[idx_v.at[pl.ds(cc * CH, CH)]], gb, gs)

        def writeback(c, gb):
            pltpu.sync_copy(gb, out_hbm.at[pl.ds(r0 + c * CH, CH)])

        gather_start(0, gb0, gs0)

        def body(s, carry):
            c = 2 * s
            gather_start(c + 1, gb1, gs1)
            pltpu.make_async_copy(
                table_hbm.at[pl.ds(0, CH)], gb0, gs0).wait()
            writeback(c, gb0)
            gather_start(c + 2, gb0, gs0)
            pltpu.make_async_copy(
                table_hbm.at[pl.ds(0, CH)], gb1, gs1).wait()
            writeback(c + 1, gb1)
            return carry

        lax.fori_loop(0, n_chunks // 2, body, 0)
        pltpu.make_async_copy(table_hbm.at[pl.ds(0, CH)], gb0, gs0).wait()
        if n_chunks % 2:
            writeback(n_chunks - 1, gb0)

    return k


def kernel(inputs, embedding_weight):
    B, H = inputs.shape
    table_p = jnp.pad(embedding_weight, ((0, 0), (0, DP - D)))
    idx_p = jnp.pad(inputs.astype(jnp.int32),
                    ((0, 0), (0, HP - H)), mode="edge").reshape(B * HP)
    out_f = _gather_kernel(B * HP)(idx_p, table_p)
    return out_f.reshape(B, HP, DP)[:, :H, :D]
